# baseline (speedup 1.0000x reference)
import jax
import jax.numpy as jnp
from jax import lax
from jax.experimental import pallas as pl
from jax.experimental.pallas import tpu as pltpu
from jax.experimental.pallas import tpu_sc as plsc

_NC = 2
_NS = 16
_NW = _NC * _NS

_CH = 128
_SLOTS = 8
_DEPTH = 4


def _sc_gather(table, idx2d, seg_ch0, seg_rows):
    w = table.shape[1]
    dt = table.dtype
    nch_w = (seg_rows // _CH) // _NW

    def body(table_hbm, idx_hbm, out_hbm, idx_v, rows_v, gsem, ssem):
        cid = lax.axis_index("c")
        sid = lax.axis_index("s")
        wid = sid * _NC + cid
        base = wid * nch_w
        pltpu.sync_copy(idx_hbm.at[pl.ds(seg_ch0 + base, nch_w)], idx_v)
        for j in range(_DEPTH):
            pltpu.async_copy(table_hbm.at[idx_v.at[j]], rows_v.at[j], gsem)

        def it(i, carry):
            slot = lax.rem(i, _SLOTS)
            pltpu.make_async_copy(
                table_hbm.at[idx_v.at[i]], rows_v.at[slot], gsem).wait()
            pltpu.async_copy(
                rows_v.at[slot],
                out_hbm.at[pl.ds((base + i) * _CH, _CH)], ssem)

            @pl.when(i >= _DEPTH)
            def _():
                pltpu.make_async_copy(
                    rows_v.at[lax.rem(i - _DEPTH, _SLOTS)],
                    out_hbm.at[pl.ds((base + i - _DEPTH) * _CH, _CH)],
                    ssem).wait()

            @pl.when(i + _DEPTH < nch_w)
            def _():
                pltpu.async_copy(
                    table_hbm.at[idx_v.at[i + _DEPTH]],
                    rows_v.at[lax.rem(i + _DEPTH, _SLOTS)], gsem)
            return carry

        lax.fori_loop(0, nch_w, it, 0)
        for j in range(_DEPTH):
            pltpu.make_async_copy(
                rows_v.at[j],
                out_hbm.at[pl.ds(base * _CH, _CH)], ssem).wait()

    mesh = plsc.VectorSubcoreMesh(core_axis_name="c", subcore_axis_name="s")
    return pl.kernel(
        body,
        out_type=jax.ShapeDtypeStruct((seg_rows, w), dt),
        mesh=mesh,
        scratch_types=[
            pltpu.VMEM((nch_w, _CH), jnp.int32),
            pltpu.VMEM((_SLOTS, _CH, w), dt),
            pltpu.SemaphoreType.DMA,
            pltpu.SemaphoreType.DMA,
        ],
        compiler_params=pltpu.CompilerParams(use_tc_tiling_on_sc=False),
    )(table, idx2d)


def _mlp_body(x_ref, w1t, b1, w2t, b2, w3t, b3, m1t, bm1, e_ref, o_ref):
    x = x_ref[0]
    f32 = jnp.float32
    h = jnp.maximum(jnp.dot(x, w1t[...],
                            preferred_element_type=f32) + b1[...], 0.0)
    h = jnp.maximum(jnp.dot(h, w2t[...],
                            preferred_element_type=f32) + b2[...], 0.0)
    lg = jnp.maximum(jnp.dot(h, w3t[...],
                             preferred_element_type=f32) + b3[...], 0.0)
    m = jnp.max(lg, axis=1, keepdims=True)
    ex = jnp.exp(lg - m)
    a = ex / jnp.sum(ex, axis=1, keepdims=True)
    aw = jnp.dot(a.astype(jnp.bfloat16), e_ref[...],
                 preferred_element_type=f32)
    y = jnp.dot(x * aw, m1t[...],
                preferred_element_type=f32) + bm1[...]
    o_ref[...] = jnp.maximum(y, 0.0).T[None]


def _tc_mlp(x, w1t, b1, w2t, b2, w3t, b3, m1t, bm1, e, tile):
    bs, M, KC = x.shape
    H = w1t.shape[1]
    K = w3t.shape[1]
    CO = m1t.shape[1]
    grid = (bs, M // tile)

    def full(shape):
        return pl.BlockSpec(shape, lambda t, i: (0,) * len(shape))

    return pl.pallas_call(
        _mlp_body,
        grid=grid,
        in_specs=[
            pl.BlockSpec((1, tile, KC), lambda t, i: (t, i, 0)),
            full((KC, H)), full((1, H)),
            full((H, H)), full((1, H)),
            full((H, K)), full((1, K)),
            full((KC, CO)), full((1, CO)),
            full((K, KC)),
        ],
        out_specs=pl.BlockSpec((1, CO, tile), lambda t, i: (t, 0, i)),
        out_shape=jax.ShapeDtypeStruct((bs, CO, M), jnp.float32),
        compiler_params=pltpu.CompilerParams(
            dimension_semantics=("arbitrary", "arbitrary")),
    )(x, w1t, b1, w2t, b2, w3t, b3, m1t, bm1, e)


def kernel(feature, idx, w1, b1, w2, b2, w3, b3, m1, bm1):
    B, C, N = feature.shape
    K = idx.shape[2]
    KC = K * C
    n_rows = B * N * K

    table = feature.transpose(0, 2, 1).reshape(B * N, C)
    idxg = (idx.astype(jnp.int32)
            + (jnp.arange(B, dtype=jnp.int32) * N)[:, None, None])
    idx2d = idxg.reshape(n_rows // _CH, _CH)

    e = jnp.kron(jnp.eye(K, dtype=jnp.float32),
                 jnp.ones((1, C), jnp.float32)).astype(jnp.bfloat16)

    S = 1
    bs = B // S
    seg_rows = n_rows // S
    seg_ch = seg_rows // _CH
    ys = []
    for s in range(S):
        grouped = _sc_gather(table, idx2d, s * seg_ch, seg_rows)
        x = grouped.reshape(bs, N, KC)
        ys.append(_tc_mlp(x, w1.T, b1[None, :], w2.T, b2[None, :],
                          w3.T, b3[None, :], m1.T, bm1[None, :], e,
                          tile=512))
    return jnp.concatenate(ys, axis=0)


# speedup vs baseline: 1.0817x; 1.0817x over previous
import jax
import jax.numpy as jnp
from jax import lax
from jax.experimental import pallas as pl
from jax.experimental.pallas import tpu as pltpu
from jax.experimental.pallas import tpu_sc as plsc

_NC = 2
_NS = 16
_NW = _NC * _NS

_CH = 128
_SLOTS = 8
_DEPTH = 6


def _sc_gather(table, idx2d, seg_ch0, seg_rows):
    w = table.shape[1]
    dt = table.dtype
    nch_w = (seg_rows // _CH) // _NW

    def body(table_hbm, idx_hbm, out_hbm, idx_v, rows_v, gsem, ssem):
        cid = lax.axis_index("c")
        sid = lax.axis_index("s")
        wid = sid * _NC + cid
        base = wid * nch_w
        pltpu.sync_copy(idx_hbm.at[pl.ds(seg_ch0 + base, nch_w)], idx_v)
        for j in range(_DEPTH):
            pltpu.async_copy(table_hbm.at[idx_v.at[j]], rows_v.at[j], gsem)

        def it(i, carry):
            slot = lax.rem(i, _SLOTS)
            pltpu.make_async_copy(
                table_hbm.at[idx_v.at[i]], rows_v.at[slot], gsem).wait()
            pltpu.async_copy(
                rows_v.at[slot],
                out_hbm.at[pl.ds((base + i) * _CH, _CH)], ssem)

            @pl.when(i >= _DEPTH)
            def _():
                pltpu.make_async_copy(
                    rows_v.at[lax.rem(i - _DEPTH, _SLOTS)],
                    out_hbm.at[pl.ds((base + i - _DEPTH) * _CH, _CH)],
                    ssem).wait()

            @pl.when(i + _DEPTH < nch_w)
            def _():
                pltpu.async_copy(
                    table_hbm.at[idx_v.at[i + _DEPTH]],
                    rows_v.at[lax.rem(i + _DEPTH, _SLOTS)], gsem)
            return carry

        lax.fori_loop(0, nch_w, it, 0)
        for j in range(_DEPTH):
            pltpu.make_async_copy(
                rows_v.at[j],
                out_hbm.at[pl.ds(base * _CH, _CH)], ssem).wait()

    mesh = plsc.VectorSubcoreMesh(core_axis_name="c", subcore_axis_name="s")
    return pl.kernel(
        body,
        out_type=jax.ShapeDtypeStruct((seg_rows, w), dt),
        mesh=mesh,
        scratch_types=[
            pltpu.VMEM((nch_w, _CH), jnp.int32),
            pltpu.VMEM((_SLOTS, _CH, w), dt),
            pltpu.SemaphoreType.DMA,
            pltpu.SemaphoreType.DMA,
        ],
        compiler_params=pltpu.CompilerParams(use_tc_tiling_on_sc=False),
    )(table, idx2d)


def _mlp_body(x_ref, w1t, b1, w2t, b2, w3t, b3, m1t, bm1, e_ref, o_ref):
    x = x_ref[0]
    f32 = jnp.float32
    h = jnp.maximum(jnp.dot(x, w1t[...],
                            preferred_element_type=f32) + b1[...], 0.0)
    h = jnp.maximum(jnp.dot(h, w2t[...],
                            preferred_element_type=f32) + b2[...], 0.0)
    lg = jnp.maximum(jnp.dot(h, w3t[...],
                             preferred_element_type=f32) + b3[...], 0.0)
    m = jnp.max(lg, axis=1, keepdims=True)
    ex = jnp.exp(lg - m)
    a = ex / jnp.sum(ex, axis=1, keepdims=True)
    aw = jnp.dot(a.astype(jnp.bfloat16), e_ref[...],
                 preferred_element_type=f32)
    y = jnp.dot(x * aw, m1t[...],
                preferred_element_type=f32) + bm1[...]
    o_ref[...] = jnp.maximum(y, 0.0).T[None]


def _tc_mlp(x, w1t, b1, w2t, b2, w3t, b3, m1t, bm1, e, tile):
    bs, M, KC = x.shape
    H = w1t.shape[1]
    K = w3t.shape[1]
    CO = m1t.shape[1]
    grid = (bs, M // tile)

    def full(shape):
        return pl.BlockSpec(shape, lambda t, i: (0,) * len(shape))

    return pl.pallas_call(
        _mlp_body,
        grid=grid,
        in_specs=[
            pl.BlockSpec((1, tile, KC), lambda t, i: (t, i, 0)),
            full((KC, H)), full((1, H)),
            full((H, H)), full((1, H)),
            full((H, K)), full((1, K)),
            full((KC, CO)), full((1, CO)),
            full((K, KC)),
        ],
        out_specs=pl.BlockSpec((1, CO, tile), lambda t, i: (t, 0, i)),
        out_shape=jax.ShapeDtypeStruct((bs, CO, M), jnp.float32),
        compiler_params=pltpu.CompilerParams(
            dimension_semantics=("arbitrary", "arbitrary")),
    )(x, w1t, b1, w2t, b2, w3t, b3, m1t, bm1, e)


def kernel(feature, idx, w1, b1, w2, b2, w3, b3, m1, bm1):
    B, C, N = feature.shape
    K = idx.shape[2]
    KC = K * C
    n_rows = B * N * K

    table = feature.transpose(0, 2, 1).reshape(B * N, C)
    idxg = (idx.astype(jnp.int32)
            + (jnp.arange(B, dtype=jnp.int32) * N)[:, None, None])
    idx2d = idxg.reshape(n_rows // _CH, _CH)

    e = jnp.kron(jnp.eye(K, dtype=jnp.float32),
                 jnp.ones((1, C), jnp.float32)).astype(jnp.bfloat16)

    S = 2
    bs = B // S
    seg_rows = n_rows // S
    seg_ch = seg_rows // _CH
    ys = []
    for s in range(S):
        grouped = _sc_gather(table, idx2d, s * seg_ch, seg_rows)
        x = grouped.reshape(bs, N, KC)
        ys.append(_tc_mlp(x, w1.T, b1[None, :], w2.T, b2[None, :],
                          w3.T, b3[None, :], m1.T, bm1[None, :], e,
                          tile=1024))
    return jnp.concatenate(ys, axis=0)


# speedup vs baseline: 1.1247x; 1.0397x over previous
import jax
import jax.numpy as jnp
from jax import lax
from jax.experimental import pallas as pl
from jax.experimental.pallas import tpu as pltpu
from jax.experimental.pallas import tpu_sc as plsc

_NC = 2
_NS = 16
_NW = _NC * _NS

_CH = 128
_SLOTS = 8
_DEPTH = 6


def _sc_gather(table, idx2d, seg_ch0, seg_rows):
    w = table.shape[1]
    dt = table.dtype
    nch_w = (seg_rows // _CH) // _NW

    def body(table_hbm, idx_hbm, out_hbm, idx_v, rows_v, gsem, ssem):
        cid = lax.axis_index("c")
        sid = lax.axis_index("s")
        wid = sid * _NC + cid
        base = wid * nch_w
        pltpu.sync_copy(idx_hbm.at[pl.ds(seg_ch0 + base, nch_w)], idx_v)
        for j in range(_DEPTH):
            pltpu.async_copy(table_hbm.at[idx_v.at[j]], rows_v.at[j], gsem)

        def it(i, carry):
            slot = lax.rem(i, _SLOTS)
            pltpu.make_async_copy(
                table_hbm.at[idx_v.at[i]], rows_v.at[slot], gsem).wait()
            pltpu.async_copy(
                rows_v.at[slot],
                out_hbm.at[pl.ds((base + i) * _CH, _CH)], ssem)

            @pl.when(i >= _DEPTH)
            def _():
                pltpu.make_async_copy(
                    rows_v.at[lax.rem(i - _DEPTH, _SLOTS)],
                    out_hbm.at[pl.ds((base + i - _DEPTH) * _CH, _CH)],
                    ssem).wait()

            @pl.when(i + _DEPTH < nch_w)
            def _():
                pltpu.async_copy(
                    table_hbm.at[idx_v.at[i + _DEPTH]],
                    rows_v.at[lax.rem(i + _DEPTH, _SLOTS)], gsem)
            return carry

        lax.fori_loop(0, nch_w, it, 0)
        for j in range(_DEPTH):
            pltpu.make_async_copy(
                rows_v.at[j],
                out_hbm.at[pl.ds(base * _CH, _CH)], ssem).wait()

    mesh = plsc.VectorSubcoreMesh(core_axis_name="c", subcore_axis_name="s")
    return pl.kernel(
        body,
        out_type=jax.ShapeDtypeStruct((seg_rows, w), dt),
        mesh=mesh,
        scratch_types=[
            pltpu.VMEM((nch_w, _CH), jnp.int32),
            pltpu.VMEM((_SLOTS, _CH, w), dt),
            pltpu.SemaphoreType.DMA,
            pltpu.SemaphoreType.DMA,
        ],
        compiler_params=pltpu.CompilerParams(use_tc_tiling_on_sc=False),
    )(table, idx2d)


def _mlp_body(x_ref, w1t, b1, w2t, b2, w3t, b3, m1t, bm1, e_ref, o_ref):
    x = x_ref[0]
    f32 = jnp.float32
    h = jnp.maximum(jnp.dot(x, w1t[...],
                            preferred_element_type=f32) + b1[...], 0.0)
    h = jnp.maximum(jnp.dot(h, w2t[...],
                            preferred_element_type=f32) + b2[...], 0.0)
    lg = jnp.maximum(jnp.dot(h, w3t[...],
                             preferred_element_type=f32) + b3[...], 0.0)
    m = jnp.max(lg, axis=1, keepdims=True)
    ex = jnp.exp(lg - m)
    a = ex / jnp.sum(ex, axis=1, keepdims=True)
    aw = jnp.dot(a.astype(jnp.bfloat16), e_ref[...],
                 preferred_element_type=f32)
    y = jnp.dot(x * aw, m1t[...],
                preferred_element_type=f32) + bm1[...]
    o_ref[...] = jnp.maximum(y, 0.0).T[None]


def _tc_mlp(x, w1t, b1, w2t, b2, w3t, b3, m1t, bm1, e, tile):
    bs, M, KC = x.shape
    H = w1t.shape[1]
    K = w3t.shape[1]
    CO = m1t.shape[1]
    grid = (bs, M // tile)

    def full(shape):
        return pl.BlockSpec(shape, lambda t, i: (0,) * len(shape))

    return pl.pallas_call(
        _mlp_body,
        grid=grid,
        in_specs=[
            pl.BlockSpec((1, tile, KC), lambda t, i: (t, i, 0)),
            full((KC, H)), full((1, H)),
            full((H, H)), full((1, H)),
            full((H, K)), full((1, K)),
            full((KC, CO)), full((1, CO)),
            full((K, KC)),
        ],
        out_specs=pl.BlockSpec((1, CO, tile), lambda t, i: (t, 0, i)),
        out_shape=jax.ShapeDtypeStruct((bs, CO, M), jnp.float32),
        compiler_params=pltpu.CompilerParams(
            dimension_semantics=("arbitrary", "arbitrary")),
    )(x, w1t, b1, w2t, b2, w3t, b3, m1t, bm1, e)


def kernel(feature, idx, w1, b1, w2, b2, w3, b3, m1, bm1):
    B, C, N = feature.shape
    K = idx.shape[2]
    KC = K * C
    n_rows = B * N * K

    table = feature.transpose(0, 2, 1).reshape(B * N, C)
    idxg = (idx.astype(jnp.int32)
            + (jnp.arange(B, dtype=jnp.int32) * N)[:, None, None])
    idx2d = idxg.reshape(n_rows // _CH, _CH)

    e = jnp.kron(jnp.eye(K, dtype=jnp.float32),
                 jnp.ones((1, C), jnp.float32)).astype(jnp.bfloat16)

    S = 2
    bs = B // S
    seg_rows = n_rows // S
    seg_ch = seg_rows // _CH
    ys = []
    for s in range(S):
        grouped = _sc_gather(table, idx2d, s * seg_ch, seg_rows)
        x = grouped.reshape(bs, N, KC)
        ys.append(_tc_mlp(x, w1.T, b1[None, :], w2.T, b2[None, :],
                          w3.T, b3[None, :], m1.T, bm1[None, :], e,
                          tile=2048))
    return jnp.concatenate(ys, axis=0)


# speedup vs baseline: 1.1365x; 1.0105x over previous
import jax
import jax.numpy as jnp
from jax import lax
from jax.experimental import pallas as pl
from jax.experimental.pallas import tpu as pltpu
from jax.experimental.pallas import tpu_sc as plsc

_NC = 2
_NS = 16
_NW = _NC * _NS

_CH = 128
_SLOTS = 8
_DEPTH = 6


def _sc_gather(table, idx2d, seg_ch0, seg_rows):
    w = table.shape[1]
    dt = table.dtype
    nch_w = (seg_rows // _CH) // _NW

    def body(table_hbm, idx_hbm, out_hbm, idx_v, rows_v, gsem, ssem):
        cid = lax.axis_index("c")
        sid = lax.axis_index("s")
        wid = sid * _NC + cid
        base = wid * nch_w
        pltpu.sync_copy(idx_hbm.at[pl.ds(seg_ch0 + base, nch_w)], idx_v)
        for j in range(_DEPTH):
            pltpu.async_copy(table_hbm.at[idx_v.at[j]], rows_v.at[j], gsem)

        def it(i, carry):
            slot = lax.rem(i, _SLOTS)
            pltpu.make_async_copy(
                table_hbm.at[idx_v.at[i]], rows_v.at[slot], gsem).wait()
            pltpu.async_copy(
                rows_v.at[slot],
                out_hbm.at[pl.ds((base + i) * _CH, _CH)], ssem)

            @pl.when(i >= _DEPTH)
            def _():
                pltpu.make_async_copy(
                    rows_v.at[lax.rem(i - _DEPTH, _SLOTS)],
                    out_hbm.at[pl.ds((base + i - _DEPTH) * _CH, _CH)],
                    ssem).wait()

            @pl.when(i + _DEPTH < nch_w)
            def _():
                pltpu.async_copy(
                    table_hbm.at[idx_v.at[i + _DEPTH]],
                    rows_v.at[lax.rem(i + _DEPTH, _SLOTS)], gsem)
            return carry

        lax.fori_loop(0, nch_w, it, 0)
        for j in range(_DEPTH):
            pltpu.make_async_copy(
                rows_v.at[j],
                out_hbm.at[pl.ds(base * _CH, _CH)], ssem).wait()

    mesh = plsc.VectorSubcoreMesh(core_axis_name="c", subcore_axis_name="s")
    return pl.kernel(
        body,
        out_type=jax.ShapeDtypeStruct((seg_rows, w), dt),
        mesh=mesh,
        scratch_types=[
            pltpu.VMEM((nch_w, _CH), jnp.int32),
            pltpu.VMEM((_SLOTS, _CH, w), dt),
            pltpu.SemaphoreType.DMA,
            pltpu.SemaphoreType.DMA,
        ],
        compiler_params=pltpu.CompilerParams(use_tc_tiling_on_sc=False),
    )(table, idx2d)


def _mlp_body(x_ref, w1t, b1, w2t, b2, w3t, b3, m1t, bm1, e_ref, o_ref):
    x = x_ref[0]
    f32 = jnp.float32
    h = jnp.maximum(jnp.dot(x, w1t[...],
                            preferred_element_type=f32) + b1[...], 0.0)
    h = jnp.maximum(jnp.dot(h, w2t[...],
                            preferred_element_type=f32) + b2[...], 0.0)
    lg = jnp.maximum(jnp.dot(h, w3t[...],
                             preferred_element_type=f32) + b3[...], 0.0)
    m = jnp.max(lg, axis=1, keepdims=True)
    ex = jnp.exp(lg - m)
    a = ex / jnp.sum(ex, axis=1, keepdims=True)
    aw = jnp.dot(a.astype(jnp.bfloat16), e_ref[...],
                 preferred_element_type=f32)
    y = jnp.dot(x * aw, m1t[...],
                preferred_element_type=f32) + bm1[...]
    o_ref[...] = jnp.maximum(y, 0.0).T[None]


def _tc_mlp(x, w1t, b1, w2t, b2, w3t, b3, m1t, bm1, e, tile):
    bs, M, KC = x.shape
    H = w1t.shape[1]
    K = w3t.shape[1]
    CO = m1t.shape[1]
    grid = (bs, M // tile)

    def full(shape):
        return pl.BlockSpec(shape, lambda t, i: (0,) * len(shape))

    return pl.pallas_call(
        _mlp_body,
        grid=grid,
        in_specs=[
            pl.BlockSpec((1, tile, KC), lambda t, i: (t, i, 0)),
            full((KC, H)), full((1, H)),
            full((H, H)), full((1, H)),
            full((H, K)), full((1, K)),
            full((KC, CO)), full((1, CO)),
            full((K, KC)),
        ],
        out_specs=pl.BlockSpec((1, CO, tile), lambda t, i: (t, 0, i)),
        out_shape=jax.ShapeDtypeStruct((bs, CO, M), jnp.float32),
        compiler_params=pltpu.CompilerParams(
            dimension_semantics=("arbitrary", "arbitrary")),
    )(x, w1t, b1, w2t, b2, w3t, b3, m1t, bm1, e)


def kernel(feature, idx, w1, b1, w2, b2, w3, b3, m1, bm1):
    B, C, N = feature.shape
    K = idx.shape[2]
    KC = K * C
    n_rows = B * N * K

    table = feature.transpose(0, 2, 1).reshape(B * N, C)
    idxg = (idx.astype(jnp.int32)
            + (jnp.arange(B, dtype=jnp.int32) * N)[:, None, None])
    idx2d = idxg.reshape(n_rows // _CH, _CH)

    e = jnp.kron(jnp.eye(K, dtype=jnp.float32),
                 jnp.ones((1, C), jnp.float32)).astype(jnp.bfloat16)

    S = 2
    bs = B // S
    seg_rows = n_rows // S
    seg_ch = seg_rows // _CH
    ys = []
    for s in range(S):
        grouped = _sc_gather(table, idx2d, s * seg_ch, seg_rows)
        x = grouped.reshape(bs, N, KC)
        ys.append(_tc_mlp(x, w1.T, b1[None, :], w2.T, b2[None, :],
                          w3.T, b3[None, :], m1.T, bm1[None, :], e,
                          tile=4096))
    return jnp.concatenate(ys, axis=0)
